# initial kernel scaffold (unmeasured)
import jax
import jax.numpy as jnp
from jax import lax
from jax.experimental import pallas as pl
from jax.experimental.pallas import tpu as pltpu


def kernel(
    x,
):
    def body(*refs):
        pass

    out_shape = jax.ShapeDtypeStruct(..., jnp.float32)
    return pl.pallas_call(body, out_shape=out_shape)(...)


# baseline (device time: 63836 ns/iter reference)
import jax
import jax.numpy as jnp
from jax import lax
from jax.experimental import pallas as pl
from jax.experimental.pallas import tpu as pltpu

N_DEV = 8
M = 512
CHUNK = 512


def kernel(x):
    def body(x_ref, out_ref, comm_ref, send_sems, recv_sems):
        my = lax.axis_index("i")
        right = lax.rem(my + 1, N_DEV)

        def my_chunk(j):
            return x_ref[0, :, pl.ds(j * CHUNK, CHUNK)].astype(jnp.bfloat16)

        comm_ref[0] = my_chunk(lax.rem(my - 1 + N_DEV, N_DEV))
        for h in range(N_DEV - 1):
            rdma = pltpu.make_async_remote_copy(
                src_ref=comm_ref.at[h],
                dst_ref=comm_ref.at[h + 1],
                send_sem=send_sems.at[h],
                recv_sem=recv_sems.at[h],
                device_id=(right,),
                device_id_type=pl.DeviceIdType.MESH,
            )
            rdma.start()
            rdma.wait()
            j = lax.rem(my - 2 - h + 2 * N_DEV, N_DEV)
            comm_ref[h + 1] = comm_ref[h + 1] + my_chunk(j)
        out_ref[:, :] = comm_ref[N_DEV - 1]

    return pl.pallas_call(
        body,
        out_shape=jax.ShapeDtypeStruct((M, CHUNK), jnp.bfloat16),
        in_specs=[pl.BlockSpec(memory_space=pltpu.VMEM)],
        out_specs=pl.BlockSpec(memory_space=pltpu.VMEM),
        scratch_shapes=[
            pltpu.VMEM((N_DEV, M, CHUNK), jnp.bfloat16),
            pltpu.SemaphoreType.DMA((N_DEV - 1,)),
            pltpu.SemaphoreType.DMA((N_DEV - 1,)),
        ],
    )(x)


# device time: 31341 ns/iter; 2.0368x vs baseline; 2.0368x over previous
import jax
import jax.numpy as jnp
from jax import lax
from jax.experimental import pallas as pl
from jax.experimental.pallas import tpu as pltpu

N_DEV = 8
M = 512
CHUNK = 512

SCHEDULES = [
    ("A", ("x", "y", "z"), 0, 176),
    ("B", ("y", "z", "x"), 176, 176),
    ("C", ("z", "x", "y"), 352, 160),
]
XOR = {"x": 1, "y": 3, "z": 4}


def _snake_pos(cx, cy, cz):
    return cz * 4 + [[0, 3], [1, 2]][cx][cy]


def kernel(x):
    def body(x_ref, out_ref, *scr):
        bufs = [scr[4 * i] for i in range(3)]
        recvs = [scr[4 * i + 1 : 4 * i + 4] for i in range(3)]
        send_sems, recv_sems = scr[12], scr[13]

        my = lax.axis_index("i")
        cx = (my & 1) ^ ((my >> 1) & 1)
        cy = (my >> 1) & 1
        cz = (my >> 2) & 1
        coord = {"x": cx, "y": cy, "z": cz}

        rdmas = [[None] * 3 for _ in range(3)]
        kept = [None] * 3

        def pack(si):
            _, order, r0, rows = SCHEDULES[si]
            for s in range(8):
                b = (s >> 2 & 1, s >> 1 & 1, s & 1)
                c = dict(zip(order, b))
                p = _snake_pos(c["x"], c["y"], c["z"])
                bufs[si][s] = x_ref[
                    0, r0 : r0 + rows, p * CHUNK : (p + 1) * CHUNK
                ].astype(jnp.bfloat16)

        def start_step(si, k):
            _, order, _, _ = SCHEDULES[si]
            ax = order[k]
            c = coord[ax]
            size = 4 >> k
            base, _ = kept[si] if k else (0, 8)
            kept[si] = (base + c * size, size)
            send_base = base + (1 - c) * size
            rdma = pltpu.make_async_remote_copy(
                src_ref=bufs[si].at[pl.ds(send_base, size)],
                dst_ref=recvs[si][k],
                send_sem=send_sems.at[si, k],
                recv_sem=recv_sems.at[si, k],
                device_id=(my ^ XOR[ax],),
                device_id_type=pl.DeviceIdType.MESH,
            )
            rdma.start()
            rdmas[si][k] = rdma

        def wait_add(si, k):
            rdmas[si][k].wait()
            base, size = kept[si]
            bufs[si][pl.ds(base, size)] = (
                bufs[si][pl.ds(base, size)] + recvs[si][k][:, :, :]
            )

        for si in range(3):
            pack(si)
            start_step(si, 0)
        for k in range(2):
            for si in range(3):
                wait_add(si, k)
                start_step(si, k + 1)
        for si in range(3):
            wait_add(si, 2)
            _, _, r0, rows = SCHEDULES[si]
            base, _ = kept[si]
            out_ref[r0 : r0 + rows, :] = bufs[si][base]

    scratch_shapes = []
    for _, _, _, rows in SCHEDULES:
        scratch_shapes.append(pltpu.VMEM((8, rows, CHUNK), jnp.bfloat16))
        for k in range(3):
            scratch_shapes.append(
                pltpu.VMEM((4 >> k, rows, CHUNK), jnp.bfloat16)
            )
    scratch_shapes.append(pltpu.SemaphoreType.DMA((3, 3)))
    scratch_shapes.append(pltpu.SemaphoreType.DMA((3, 3)))

    return pl.pallas_call(
        body,
        out_shape=jax.ShapeDtypeStruct((M, CHUNK), jnp.bfloat16),
        in_specs=[pl.BlockSpec(memory_space=pltpu.VMEM)],
        out_specs=pl.BlockSpec(memory_space=pltpu.VMEM),
        scratch_shapes=scratch_shapes,
    )(x)


# device time: 26495 ns/iter; 2.4094x vs baseline; 1.1829x over previous
import jax
import jax.numpy as jnp
from jax import lax
from jax.experimental import pallas as pl
from jax.experimental.pallas import tpu as pltpu

N_DEV = 8
M = 512
CHUNK = 512

SCHEDULES = [
    ("A", ("x", "y", "z"), 0, 176),
    ("B", ("y", "z", "x"), 176, 176),
    ("C", ("z", "x", "y"), 352, 160),
]
XOR = {"x": 1, "y": 3, "z": 4}


def _snake_pos(cx, cy, cz):
    return cz * 4 + [[0, 3], [1, 2]][cx][cy]


def kernel(x):
    def body(x_ref, out_ref, *scr):
        bufs = [scr[4 * i] for i in range(3)]
        recvs = [scr[4 * i + 1 : 4 * i + 4] for i in range(3)]
        send_sems, recv_sems = scr[12], scr[13]

        my = lax.axis_index("i")
        cx = (my & 1) ^ ((my >> 1) & 1)
        cy = (my >> 1) & 1
        cz = (my >> 2) & 1
        coord = {"x": cx, "y": cy, "z": cz}

        barrier = pltpu.get_barrier_semaphore()
        for ax in ("x", "y", "z"):
            pl.semaphore_signal(
                barrier,
                inc=1,
                device_id=(my ^ XOR[ax],),
                device_id_type=pl.DeviceIdType.MESH,
            )
        pl.semaphore_wait(barrier, 3)

        rdmas = [[None] * 3 for _ in range(3)]
        kept = [None] * 3

        def pack_slots(si, slots):
            _, order, r0, rows = SCHEDULES[si]
            for s in slots:
                b = (s >> 2 & 1, s >> 1 & 1, s & 1)
                c = dict(zip(order, b))
                p = _snake_pos(c["x"], c["y"], c["z"])
                bufs[si][s] = x_ref[
                    0, r0 : r0 + rows, p * CHUNK : (p + 1) * CHUNK
                ].astype(jnp.bfloat16)

        def start_step(si, k):
            _, order, _, _ = SCHEDULES[si]
            c = coord[order[k]]
            size = 4 >> k
            base, _ = kept[si] if k else (0, 8)
            kept[si] = (base + c * size, size)
            send_base = base + (1 - c) * size
            rdma = pltpu.make_async_remote_copy(
                src_ref=bufs[si].at[pl.ds(send_base, size)],
                dst_ref=recvs[si][k],
                send_sem=send_sems.at[si, k],
                recv_sem=recv_sems.at[si, k],
                device_id=(my ^ XOR[order[k]],),
                device_id_type=pl.DeviceIdType.MESH,
            )
            rdma.start()
            rdmas[si][k] = rdma

        def add_sub(si, k, base, off, n):
            bufs[si][pl.ds(base + off, n)] = (
                bufs[si][pl.ds(base + off, n)] + recvs[si][k][pl.ds(off, n)]
            )

        for si in range(3):
            c1 = coord[SCHEDULES[si][1][0]]
            pl.when(c1 == 0)(lambda si=si: pack_slots(si, range(4, 8)))
            pl.when(c1 == 1)(lambda si=si: pack_slots(si, range(0, 4)))
            start_step(si, 0)
            pl.when(c1 == 0)(lambda si=si: pack_slots(si, range(0, 4)))
            pl.when(c1 == 1)(lambda si=si: pack_slots(si, range(4, 8)))
        for k in range(2):
            for si in range(3):
                rdmas[si][k].wait()
                base, size = kept[si]
                h = size // 2
                cn = coord[SCHEDULES[si][1][k + 1]]
                add_sub(si, k, base, (1 - cn) * h, h)
                start_step(si, k + 1)
                add_sub(si, k, base, cn * h, h)
        for si in range(3):
            rdmas[si][2].wait()
            base, _ = kept[si]
            add_sub(si, 2, base, 0, 1)
            _, _, r0, rows = SCHEDULES[si]
            out_ref[r0 : r0 + rows, :] = bufs[si][base]

    scratch_shapes = []
    for _, _, _, rows in SCHEDULES:
        scratch_shapes.append(pltpu.VMEM((8, rows, CHUNK), jnp.bfloat16))
        for k in range(3):
            scratch_shapes.append(
                pltpu.VMEM((4 >> k, rows, CHUNK), jnp.bfloat16)
            )
    scratch_shapes.append(pltpu.SemaphoreType.DMA((3, 3)))
    scratch_shapes.append(pltpu.SemaphoreType.DMA((3, 3)))

    return pl.pallas_call(
        body,
        out_shape=jax.ShapeDtypeStruct((M, CHUNK), jnp.bfloat16),
        in_specs=[pl.BlockSpec(memory_space=pltpu.VMEM)],
        out_specs=pl.BlockSpec(memory_space=pltpu.VMEM),
        scratch_shapes=scratch_shapes,
        compiler_params=pltpu.CompilerParams(collective_id=0),
    )(x)


# device time: 23413 ns/iter; 2.7265x vs baseline; 1.1316x over previous
import jax
import jax.numpy as jnp
from jax import lax
from jax.experimental import pallas as pl
from jax.experimental.pallas import tpu as pltpu

N_DEV = 8
M = 512
CHUNK = 512

SCHEDULES = [
    ("A0", ("x", "y", "z"), 0, 96),
    ("B0", ("y", "z", "x"), 176, 96),
    ("C0", ("z", "x", "y"), 352, 80),
    ("A1", ("x", "y", "z"), 96, 80),
    ("B1", ("y", "z", "x"), 272, 80),
    ("C1", ("z", "x", "y"), 432, 80),
]
NS = len(SCHEDULES)
XOR = {"x": 1, "y": 3, "z": 4}


def _snake_pos(cx, cy, cz):
    return cz * 4 + [[0, 3], [1, 2]][cx][cy]


def kernel(x):
    def body(x_ref, out_ref, *scr):
        bufs = [scr[4 * i] for i in range(NS)]
        recvs = [scr[4 * i + 1 : 4 * i + 4] for i in range(NS)]
        send_sems, recv_sems = scr[4 * NS], scr[4 * NS + 1]

        my = lax.axis_index("i")
        cx = (my & 1) ^ ((my >> 1) & 1)
        cy = (my >> 1) & 1
        cz = (my >> 2) & 1
        coord = {"x": cx, "y": cy, "z": cz}

        barrier = pltpu.get_barrier_semaphore()
        for ax in ("x", "y", "z"):
            pl.semaphore_signal(
                barrier,
                inc=1,
                device_id=(my ^ XOR[ax],),
                device_id_type=pl.DeviceIdType.MESH,
            )
        pl.semaphore_wait(barrier, 3)

        rdmas = [[None] * 3 for _ in range(NS)]
        kept = [None] * NS

        def pack_slots(si, slots):
            _, order, r0, rows = SCHEDULES[si]
            for s in slots:
                b = (s >> 2 & 1, s >> 1 & 1, s & 1)
                c = dict(zip(order, b))
                p = _snake_pos(c["x"], c["y"], c["z"])
                bufs[si][s] = x_ref[
                    0, r0 : r0 + rows, p * CHUNK : (p + 1) * CHUNK
                ].astype(jnp.bfloat16)

        def start_step(si, k):
            _, order, _, _ = SCHEDULES[si]
            c = coord[order[k]]
            size = 4 >> k
            base, _ = kept[si] if k else (0, 8)
            kept[si] = (base + c * size, size)
            send_base = base + (1 - c) * size
            rdma = pltpu.make_async_remote_copy(
                src_ref=bufs[si].at[pl.ds(send_base, size)],
                dst_ref=recvs[si][k],
                send_sem=send_sems.at[si, k],
                recv_sem=recv_sems.at[si, k],
                device_id=(my ^ XOR[order[k]],),
                device_id_type=pl.DeviceIdType.MESH,
            )
            rdma.start()
            rdmas[si][k] = rdma

        def add_sub(si, k, base, off, n):
            bufs[si][pl.ds(base + off, n)] = (
                bufs[si][pl.ds(base + off, n)] + recvs[si][k][pl.ds(off, n)]
            )

        for si in range(NS):
            c1 = coord[SCHEDULES[si][1][0]]
            pl.when(c1 == 0)(lambda si=si: pack_slots(si, range(4, 8)))
            pl.when(c1 == 1)(lambda si=si: pack_slots(si, range(0, 4)))
            start_step(si, 0)
        for si in range(NS):
            c1 = coord[SCHEDULES[si][1][0]]
            pl.when(c1 == 0)(lambda si=si: pack_slots(si, range(0, 4)))
            pl.when(c1 == 1)(lambda si=si: pack_slots(si, range(4, 8)))
        for k in range(2):
            for si in range(NS):
                rdmas[si][k].wait()
                base, size = kept[si]
                h = size // 2
                cn = coord[SCHEDULES[si][1][k + 1]]
                add_sub(si, k, base, (1 - cn) * h, h)
                start_step(si, k + 1)
                add_sub(si, k, base, cn * h, h)
        for si in range(NS):
            rdmas[si][2].wait()
            base, _ = kept[si]
            add_sub(si, 2, base, 0, 1)
            _, _, r0, rows = SCHEDULES[si]
            out_ref[r0 : r0 + rows, :] = bufs[si][base]

    scratch_shapes = []
    for _, _, _, rows in SCHEDULES:
        scratch_shapes.append(pltpu.VMEM((8, rows, CHUNK), jnp.bfloat16))
        for k in range(3):
            scratch_shapes.append(
                pltpu.VMEM((4 >> k, rows, CHUNK), jnp.bfloat16)
            )
    scratch_shapes.append(pltpu.SemaphoreType.DMA((NS, 3)))
    scratch_shapes.append(pltpu.SemaphoreType.DMA((NS, 3)))

    return pl.pallas_call(
        body,
        out_shape=jax.ShapeDtypeStruct((M, CHUNK), jnp.bfloat16),
        in_specs=[pl.BlockSpec(memory_space=pltpu.VMEM)],
        out_specs=pl.BlockSpec(memory_space=pltpu.VMEM),
        scratch_shapes=scratch_shapes,
        compiler_params=pltpu.CompilerParams(collective_id=0),
    )(x)
